# baseline (speedup 1.0000x reference)
import functools

import jax
import jax.numpy as jnp
from jax import lax
from jax.experimental import pallas as pl
from jax.experimental.pallas import tpu as pltpu
from jax.experimental.pallas import tpu_sc as plsc

EMBED = 64
EPS = 1e-5
NC = 2
NS = 16
NW = NC * NS
SEQ_PER_CHUNK = 2
LANES = 16
UNROLL = 2


def _body(table_hbm, idx_hbm, gamma_hbm, beta_hbm, out_hbm,
          idx_v, row0, row1, ob0, ob1, gb_v,
          gsem0, gsem1, wsem0, wsem1, n_chunks, seq_len):
    rowb = (row0, row1)
    outb = (ob0, ob1)
    gsem = (gsem0, gsem1)
    wsem = (wsem0, wsem1)
    ch_r = SEQ_PER_CHUNK * seq_len
    spw = n_chunks * SEQ_PER_CHUNK
    cid = lax.axis_index("c")
    sid = lax.axis_index("s")
    wid = sid * NC + cid
    sbase = wid * spw
    lane = lax.iota(jnp.int32, LANES)
    zerov = jnp.zeros((LANES,), jnp.int32)
    onev = jnp.ones((LANES,), jnp.int32)
    pltpu.sync_copy(idx_hbm.at[pl.ds(sbase, spw)], idx_v)
    pltpu.sync_copy(gamma_hbm, gb_v.at[0])
    pltpu.sync_copy(beta_hbm, gb_v.at[1])

    offs = list(range(0, ch_r - LANES + 1, LANES))
    if offs[-1] != ch_r - LANES:
        offs.append(ch_r - LANES)
    ngrp = len(offs)
    rvec = [lane + o for o in offs]

    def gather_dma(j, b, k):
        return pltpu.make_async_copy(
            table_hbm.at[idx_v.at[j * SEQ_PER_CHUNK + k]],
            rowb[b].at[pl.ds(k * seq_len, seq_len)],
            gsem[b])

    def wb_dma(j, b, k):
        return pltpu.make_async_copy(
            outb[b].at[pl.ds(k * seq_len, seq_len)],
            out_hbm.at[sbase + j * SEQ_PER_CHUNK + k],
            wsem[b])

    def compute(b):
        rows = rowb[b]
        outv = outb[b]

        def p1(i, c):
            new = list(c)
            for u in range(UNROLL):
                e = i * UNROLL + u
                evec = (lane + e) & (EMBED - 1)
                for g in range(ngrp):
                    x = plsc.load_gather(rows, [rvec[g], evec])
                    new[g] = new[g] + x
                    new[ngrp + g] = new[ngrp + g] + x * x
            return tuple(new)

        zf = jnp.zeros((LANES,), jnp.float32)
        acc = lax.fori_loop(0, EMBED // UNROLL, p1, (zf,) * (2 * ngrp))

        mean = []
        inv = []
        for g in range(ngrp):
            m = acc[g] * (1.0 / EMBED)
            v = acc[ngrp + g] * (1.0 / EMBED) - m * m
            h = v + EPS
            bits = plsc.bitcast(h, jnp.int32)
            y = plsc.bitcast(jnp.int32(0x5F3759DF) - (bits >> 1), jnp.float32)
            nh = h * (-0.5)
            for _ in range(3):
                y = y * (1.5 + nh * y * y)
            mean.append(m)
            inv.append(y)

        def p2(i, c):
            for u in range(UNROLL):
                e = i * UNROLL + u
                evec = (lane + e) & (EMBED - 1)
                ge = plsc.load_gather(gb_v, [zerov, evec])
                be = plsc.load_gather(gb_v, [onev, evec])
                for g in range(ngrp):
                    x = plsc.load_gather(rows, [rvec[g], evec])
                    t = (x - mean[g]) * inv[g]
                    plsc.store_scatter(outv, [rvec[g], evec], t * ge + be)
            return c

        lax.fori_loop(0, EMBED // UNROLL, p2, 0)

    for k in range(SEQ_PER_CHUNK):
        gather_dma(0, 0, k).start()
        gather_dma(1, 1, k).start()

    def outer(jo, carry):
        for b in range(2):
            j = jo * 2 + b
            for k in range(SEQ_PER_CHUNK):
                gather_dma(j, b, k).wait()
            compute(b)

            @pl.when(j + 2 < n_chunks)
            def _():
                for k in range(SEQ_PER_CHUNK):
                    gather_dma(j + 2, b, k).start()

            @pl.when(j >= 2)
            def _():
                for k in range(SEQ_PER_CHUNK):
                    wb_dma(j - 2, b, k).wait()

            for k in range(SEQ_PER_CHUNK):
                wb_dma(j, b, k).start()
        return carry

    lax.fori_loop(0, n_chunks // 2, outer, 0)
    for k in range(SEQ_PER_CHUNK):
        wb_dma(n_chunks - 2, 0, k).wait()
        wb_dma(n_chunks - 1, 1, k).wait()


def kernel(input_ids, table, gamma, beta):
    b, l = input_ids.shape
    seq_per_w = b // NW
    n_chunks = seq_per_w // SEQ_PER_CHUNK
    ch_r = SEQ_PER_CHUNK * l
    assert b % NW == 0 and seq_per_w % 2 == 0
    mesh = plsc.VectorSubcoreMesh(core_axis_name="c", subcore_axis_name="s")
    f = pl.kernel(
        functools.partial(_body, n_chunks=n_chunks, seq_len=l),
        mesh=mesh,
        compiler_params=pltpu.CompilerParams(
            needs_layout_passes=False, use_tc_tiling_on_sc=False
        ),
        out_type=jax.ShapeDtypeStruct((b, l, EMBED), jnp.float32),
        scratch_types=[
            pltpu.VMEM((seq_per_w, l), jnp.int32),
            pltpu.VMEM((ch_r, EMBED), jnp.float32),
            pltpu.VMEM((ch_r, EMBED), jnp.float32),
            pltpu.VMEM((ch_r, EMBED), jnp.float32),
            pltpu.VMEM((ch_r, EMBED), jnp.float32),
            pltpu.VMEM((2, EMBED), jnp.float32),
            pltpu.SemaphoreType.DMA,
            pltpu.SemaphoreType.DMA,
            pltpu.SemaphoreType.DMA,
            pltpu.SemaphoreType.DMA,
        ],
    )
    return f(table, input_ids.astype(jnp.int32), gamma, beta)


# speedup vs baseline: 1.0296x; 1.0296x over previous
import functools

import jax
import jax.numpy as jnp
from jax import lax
from jax.experimental import pallas as pl
from jax.experimental.pallas import tpu as pltpu
from jax.experimental.pallas import tpu_sc as plsc

EMBED = 64
EPS = 1e-5
NC = 2
NS = 16
NW = NC * NS
CH = 128
LANES = 16
NGRP = CH // LANES
UNROLL = 2


def _body(table_hbm, idx_hbm, gamma_hbm, beta_hbm, out_hbm,
          idx_v, row0, row1, ob0, ob1, gb_v,
          gsem0, gsem1, wsem0, wsem1, n_chunks):
    rowb = (row0, row1)
    outb = (ob0, ob1)
    gsem = (gsem0, gsem1)
    wsem = (wsem0, wsem1)
    cid = lax.axis_index("c")
    sid = lax.axis_index("s")
    wid = sid * NC + cid
    bpw = n_chunks * CH
    rbase = wid * bpw
    pltpu.sync_copy(idx_hbm.at[pl.ds(rbase, bpw)], idx_v)
    pltpu.sync_copy(gamma_hbm, gb_v.at[0])
    pltpu.sync_copy(beta_hbm, gb_v.at[1])

    lane = lax.iota(jnp.int32, LANES)
    zerov = jnp.zeros((LANES,), jnp.int32)
    onev = jnp.ones((LANES,), jnp.int32)
    rvec = [lane + g * LANES for g in range(NGRP)]
    rflat = [(lane + g * LANES) * EMBED for g in range(NGRP)]

    def gather_dma(j, b):
        return pltpu.make_async_copy(
            table_hbm.at[idx_v.at[pl.ds(j * CH, CH)]], rowb[b], gsem[b])

    def wb_dma(j, b):
        return pltpu.make_async_copy(
            outb[b], out_hbm.at[pl.ds((rbase + j * CH) * EMBED, CH * EMBED)],
            wsem[b])

    def compute(b):
        rows = rowb[b]
        outv = outb[b]

        def p1(i, c):
            new = list(c)
            for u in range(UNROLL):
                e = i * UNROLL + u
                evec = (lane + e) & (EMBED - 1)
                for g in range(NGRP):
                    x = plsc.load_gather(rows, [rvec[g], evec])
                    new[g] = new[g] + x
                    new[NGRP + g] = new[NGRP + g] + x * x
            return tuple(new)

        zf = jnp.zeros((LANES,), jnp.float32)
        acc = lax.fori_loop(0, EMBED // UNROLL, p1, (zf,) * (2 * NGRP))

        mean = []
        inv = []
        for g in range(NGRP):
            m = acc[g] * (1.0 / EMBED)
            v = acc[NGRP + g] * (1.0 / EMBED) - m * m
            h = v + EPS
            bits = plsc.bitcast(h, jnp.int32)
            y = plsc.bitcast(jnp.int32(0x5F3759DF) - (bits >> 1), jnp.float32)
            nh = h * (-0.5)
            for _ in range(3):
                y = y * (1.5 + nh * y * y)
            mean.append(m)
            inv.append(y)

        def p2(i, c):
            for u in range(UNROLL):
                e = i * UNROLL + u
                evec = (lane + e) & (EMBED - 1)
                ge = plsc.load_gather(gb_v, [zerov, evec])
                be = plsc.load_gather(gb_v, [onev, evec])
                for g in range(NGRP):
                    x = plsc.load_gather(rows, [rvec[g], evec])
                    t = (x - mean[g]) * inv[g]
                    plsc.store_scatter(outv, [rflat[g] + evec], t * ge + be)
            return c

        lax.fori_loop(0, EMBED // UNROLL, p2, 0)

    gather_dma(0, 0).start()
    gather_dma(1, 1).start()

    def outer(jo, carry):
        for b in range(2):
            j = jo * 2 + b
            gather_dma(j, b).wait()
            compute(b)

            @pl.when(j + 2 < n_chunks)
            def _():
                gather_dma(j + 2, b).start()

            @pl.when(j >= 2)
            def _():
                wb_dma(j - 2, b).wait()

            wb_dma(j, b).start()
        return carry

    lax.fori_loop(0, n_chunks // 2, outer, 0)
    wb_dma(n_chunks - 2, 0).wait()
    wb_dma(n_chunks - 1, 1).wait()


def kernel(input_ids, table, gamma, beta):
    b, l = input_ids.shape
    n = b * l
    assert n % (NW * CH) == 0
    n_chunks = n // (NW * CH)
    assert n_chunks % 2 == 0
    idx_flat = input_ids.reshape(-1).astype(jnp.int32)
    mesh = plsc.VectorSubcoreMesh(core_axis_name="c", subcore_axis_name="s")
    f = pl.kernel(
        functools.partial(_body, n_chunks=n_chunks),
        mesh=mesh,
        compiler_params=pltpu.CompilerParams(
            needs_layout_passes=False, use_tc_tiling_on_sc=False
        ),
        out_type=jax.ShapeDtypeStruct((n * EMBED,), jnp.float32),
        scratch_types=[
            pltpu.VMEM((n_chunks * CH,), jnp.int32),
            pltpu.VMEM((CH, EMBED), jnp.float32),
            pltpu.VMEM((CH, EMBED), jnp.float32),
            pltpu.VMEM((CH * EMBED,), jnp.float32),
            pltpu.VMEM((CH * EMBED,), jnp.float32),
            pltpu.VMEM((2, EMBED), jnp.float32),
            pltpu.SemaphoreType.DMA,
            pltpu.SemaphoreType.DMA,
            pltpu.SemaphoreType.DMA,
            pltpu.SemaphoreType.DMA,
        ],
    )
    out = f(table, idx_flat, gamma, beta)
    return out.reshape(b, l, EMBED)


# speedup vs baseline: 1.0405x; 1.0106x over previous
import functools

import jax
import jax.numpy as jnp
from jax import lax
from jax.experimental import pallas as pl
from jax.experimental.pallas import tpu as pltpu
from jax.experimental.pallas import tpu_sc as plsc

EMBED = 64
EPS = 1e-5
NC = 2
NS = 16
NW = NC * NS
CH = 128
LANES = 16
NGRP = CH // LANES
UNROLL = 4


def _body(table_hbm, idx_hbm, gamma_hbm, beta_hbm, out_hbm,
          idx_v, row0, row1, ob0, ob1, gb_v,
          gsem0, gsem1, wsem0, wsem1, n_chunks):
    rowb = (row0, row1)
    outb = (ob0, ob1)
    gsem = (gsem0, gsem1)
    wsem = (wsem0, wsem1)
    cid = lax.axis_index("c")
    sid = lax.axis_index("s")
    wid = sid * NC + cid
    bpw = n_chunks * CH
    rbase = wid * bpw
    pltpu.sync_copy(idx_hbm.at[pl.ds(rbase, bpw)], idx_v)
    pltpu.sync_copy(gamma_hbm, gb_v.at[0])
    pltpu.sync_copy(beta_hbm, gb_v.at[1])

    lane = lax.iota(jnp.int32, LANES)
    zerov = jnp.zeros((LANES,), jnp.int32)
    onev = jnp.ones((LANES,), jnp.int32)
    rvec = [lane + g * LANES for g in range(NGRP)]
    rflat = [(lane + g * LANES) * EMBED for g in range(NGRP)]

    def gather_dma(j, b):
        return pltpu.make_async_copy(
            table_hbm.at[idx_v.at[pl.ds(j * CH, CH)]], rowb[b], gsem[b])

    def wb_dma(j, b):
        return pltpu.make_async_copy(
            outb[b], out_hbm.at[pl.ds((rbase + j * CH) * EMBED, CH * EMBED)],
            wsem[b])

    def compute(b):
        rows = rowb[b]
        outv = outb[b]

        def p1(i, c):
            new = list(c)
            for u in range(UNROLL):
                e = i * UNROLL + u
                evec = (lane + e) & (EMBED - 1)
                for g in range(NGRP):
                    x = plsc.load_gather(rows, [rvec[g], evec])
                    new[g] = new[g] + x
                    new[NGRP + g] = new[NGRP + g] + x * x
            return tuple(new)

        zf = jnp.zeros((LANES,), jnp.float32)
        acc = lax.fori_loop(0, EMBED // UNROLL, p1, (zf,) * (2 * NGRP))

        mean = []
        inv = []
        for g in range(NGRP):
            m = acc[g] * (1.0 / EMBED)
            v = acc[NGRP + g] * (1.0 / EMBED) - m * m
            h = v + EPS
            bits = plsc.bitcast(h, jnp.int32)
            y = plsc.bitcast(jnp.int32(0x5F3759DF) - (bits >> 1), jnp.float32)
            nh = h * (-0.5)
            for _ in range(3):
                y = y * (1.5 + nh * y * y)
            mean.append(m)
            inv.append(y)

        def p2(i, c):
            for u in range(UNROLL):
                e = i * UNROLL + u
                evec = (lane + e) & (EMBED - 1)
                ge = plsc.load_gather(gb_v, [zerov, evec])
                be = plsc.load_gather(gb_v, [onev, evec])
                for g in range(NGRP):
                    x = plsc.load_gather(rows, [rvec[g], evec])
                    t = (x - mean[g]) * inv[g]
                    plsc.store_scatter(outv, [rflat[g] + evec], t * ge + be)
            return c

        lax.fori_loop(0, EMBED // UNROLL, p2, 0)

    gather_dma(0, 0).start()
    gather_dma(1, 1).start()

    def outer(jo, carry):
        for b in range(2):
            j = jo * 2 + b
            gather_dma(j, b).wait()
            compute(b)

            @pl.when(j + 2 < n_chunks)
            def _():
                gather_dma(j + 2, b).start()

            @pl.when(j >= 2)
            def _():
                wb_dma(j - 2, b).wait()

            wb_dma(j, b).start()
        return carry

    lax.fori_loop(0, n_chunks // 2, outer, 0)
    wb_dma(n_chunks - 2, 0).wait()
    wb_dma(n_chunks - 1, 1).wait()


def kernel(input_ids, table, gamma, beta):
    b, l = input_ids.shape
    n = b * l
    assert n % (NW * CH) == 0
    n_chunks = n // (NW * CH)
    assert n_chunks % 2 == 0
    idx_flat = input_ids.reshape(-1).astype(jnp.int32)
    mesh = plsc.VectorSubcoreMesh(core_axis_name="c", subcore_axis_name="s")
    f = pl.kernel(
        functools.partial(_body, n_chunks=n_chunks),
        mesh=mesh,
        compiler_params=pltpu.CompilerParams(
            needs_layout_passes=False, use_tc_tiling_on_sc=False
        ),
        out_type=jax.ShapeDtypeStruct((n * EMBED,), jnp.float32),
        scratch_types=[
            pltpu.VMEM((n_chunks * CH,), jnp.int32),
            pltpu.VMEM((CH, EMBED), jnp.float32),
            pltpu.VMEM((CH, EMBED), jnp.float32),
            pltpu.VMEM((CH * EMBED,), jnp.float32),
            pltpu.VMEM((CH * EMBED,), jnp.float32),
            pltpu.VMEM((2, EMBED), jnp.float32),
            pltpu.SemaphoreType.DMA,
            pltpu.SemaphoreType.DMA,
            pltpu.SemaphoreType.DMA,
            pltpu.SemaphoreType.DMA,
        ],
    )
    out = f(table, idx_flat, gamma, beta)
    return out.reshape(b, l, EMBED)
